# Initial kernel scaffold; baseline (speedup 1.0000x reference)
import jax
import jax.numpy as jnp
from jax import lax
from jax.experimental import pallas as pl
from jax.experimental.pallas import tpu as pltpu
from jax.experimental.pallas import tpu_sc as plsc

N = 10000
D = 128
E = 320000

NC = 2
NS = 16
L = 16
NW = NC * NS
EPW = E // NW
C = 80
NCHUNK = EPW // C
RPT = N // NS
DW = 16

_BA = 1000
_BC = 1000


def _tc_head(x_ref, w_ref, al_ref, ar_ref, h_ref, elr_ref):
    h = jnp.dot(x_ref[...], w_ref[...], preferred_element_type=jnp.float32)
    h_ref[...] = h
    elr_ref[0, :] = jnp.sum(h * al_ref[...], axis=1)
    elr_ref[1, :] = jnp.sum(h * ar_ref[...], axis=1)


def _tc_head_call(x, W, al, ar):
    return pl.pallas_call(
        _tc_head,
        grid=(N // _BA,),
        in_specs=[
            pl.BlockSpec((_BA, D), lambda i: (i, 0)),
            pl.BlockSpec((D, D), lambda i: (0, 0)),
            pl.BlockSpec((1, D), lambda i: (0, 0)),
            pl.BlockSpec((1, D), lambda i: (0, 0)),
        ],
        out_specs=[
            pl.BlockSpec((_BA, D), lambda i: (i, 0)),
            pl.BlockSpec((2, _BA), lambda i: (0, i)),
        ],
        out_shape=[
            jax.ShapeDtypeStruct((N, D), jnp.float32),
            jax.ShapeDtypeStruct((2, N), jnp.float32),
        ],
    )(x, W, al, ar)


def _sc_edges(h_hbm, elr_hbm, src_hbm, dst_hbm, z128_hbm, zdw_hbm,
              accp_hbm, denp_hbm,
              el_v, er_v, src_v, dst_v, w_v, rows_v, acc_sh, den_sh, sem):
    c = lax.axis_index("c")
    s = lax.axis_index("s")
    wid = c * NS + s

    pltpu.sync_copy(elr_hbm.at[0], el_v)
    pltpu.sync_copy(elr_hbm.at[1], er_v)
    pltpu.sync_copy(src_hbm.at[wid], src_v)
    pltpu.sync_copy(dst_hbm.at[wid], dst_v)
    pltpu.sync_copy(zdw_hbm.at[pl.ds(0, C)], w_v)

    rbase = s * RPT
    pltpu.sync_copy(z128_hbm.at[pl.ds(rbase, RPT)], acc_sh.at[pl.ds(rbase, RPT)])
    pltpu.sync_copy(zdw_hbm.at[pl.ds(rbase, RPT)], den_sh.at[pl.ds(rbase, RPT)])
    plsc.subcore_barrier()

    col0 = jnp.zeros((L,), jnp.int32)

    def chunk_body(j, carry):
        gather = pltpu.async_copy(
            h_hbm.at[src_v.at[pl.ds(j * C, C)]], rows_v, sem)

        def w_body(k, carry2):
            off = j * C + k * L
            srcv = src_v[pl.ds(off, L)]
            dstv = dst_v[j, pl.ds(k * L, L)]
            e = plsc.load_gather(el_v, [srcv]) + plsc.load_gather(er_v, [dstv])
            e = jnp.where(e >= 0.0, e, e * 0.2)
            w = jnp.exp(e)
            plsc.store_scatter(
                w_v, [k * L + lax.iota(jnp.int32, (L,)), col0], w)
            return carry2

        lax.fori_loop(0, C // L, w_body, 0)
        gather.wait()

        def row_body(r, carry2):
            wr = w_v[r, 0]
            for q in range(D // L):
                rows_v[r, pl.ds(q * L, L)] = rows_v[r, pl.ds(q * L, L)] * wr
            return carry2

        lax.fori_loop(0, C, row_body, 0)

        pltpu.sync_copy(rows_v, acc_sh.at[dst_v.at[j]], add=True)
        pltpu.sync_copy(w_v, den_sh.at[dst_v.at[j]], add=True)
        return carry

    lax.fori_loop(0, NCHUNK, chunk_body, 0)

    plsc.subcore_barrier()
    pltpu.sync_copy(acc_sh.at[pl.ds(rbase, RPT)],
                    accp_hbm.at[c, pl.ds(rbase, RPT)])
    pltpu.sync_copy(den_sh.at[pl.ds(rbase, RPT)],
                    denp_hbm.at[c, pl.ds(rbase, RPT)])


def _sc_edges_call(h, elr, src2, dst3, z128, zdw):
    mesh = plsc.VectorSubcoreMesh(
        core_axis_name="c", subcore_axis_name="s", num_cores=NC,
        num_subcores=NS)
    return pl.kernel(
        _sc_edges,
        out_type=[
            jax.ShapeDtypeStruct((NC, N, D), jnp.float32),
            jax.ShapeDtypeStruct((NC, N, DW), jnp.float32),
        ],
        mesh=mesh,
        scratch_types=[
            pltpu.VMEM((N,), jnp.float32),
            pltpu.VMEM((N,), jnp.float32),
            pltpu.VMEM((EPW,), jnp.int32),
            pltpu.VMEM((NCHUNK, C), jnp.int32),
            pltpu.VMEM((C, DW), jnp.float32),
            pltpu.VMEM((C, D), jnp.float32),
            pltpu.VMEM_SHARED((N, D), jnp.float32),
            pltpu.VMEM_SHARED((N, DW), jnp.float32),
            pltpu.SemaphoreType.DMA,
        ],
    )(h, elr, src2, dst3, z128, zdw)


def _tc_tail(accp_ref, denp_ref, bias_ref, g_ref, b_ref, out_ref):
    acc = accp_ref[0] + accp_ref[1]
    den = denp_ref[0, :, 0:1] + denp_ref[1, :, 0:1]
    den = jnp.where(den > 0.0, den, 1.0)
    rst = acc / den + bias_ref[...]
    mu = jnp.mean(rst, axis=1, keepdims=True)
    var = jnp.mean((rst - mu) ** 2, axis=1, keepdims=True)
    y = (rst - mu) * lax.rsqrt(var + 1e-5) * g_ref[...] + b_ref[...]
    out_ref[...] = jnp.where(y > 0.0, y, jnp.exp(y) - 1.0)


def _tc_tail_call(accp, denp, bias, ln_g, ln_b):
    return pl.pallas_call(
        _tc_tail,
        grid=(N // _BC,),
        in_specs=[
            pl.BlockSpec((NC, _BC, D), lambda i: (0, i, 0)),
            pl.BlockSpec((NC, _BC, DW), lambda i: (0, i, 0)),
            pl.BlockSpec((1, D), lambda i: (0, 0)),
            pl.BlockSpec((1, D), lambda i: (0, 0)),
            pl.BlockSpec((1, D), lambda i: (0, 0)),
        ],
        out_specs=pl.BlockSpec((_BC, D), lambda i: (i, 0)),
        out_shape=jax.ShapeDtypeStruct((N, D), jnp.float32),
    )(accp, denp, bias, ln_g, ln_b)


@jax.jit
def kernel(features, edge_index, W, attn_l, attn_r, bias, ln_g, ln_b):
    src = edge_index[0].astype(jnp.int32).reshape(NW, EPW)
    dst = edge_index[1].astype(jnp.int32).reshape(NW, NCHUNK, C)
    al = attn_l.reshape(1, D).astype(jnp.float32)
    ar = attn_r.reshape(1, D).astype(jnp.float32)
    h, elr = _tc_head_call(features, W, al, ar)
    z128 = jnp.zeros((N, D), jnp.float32)
    zdw = jnp.zeros((N, DW), jnp.float32)
    accp, denp = _sc_edges_call(h, elr, src, dst, z128, zdw)
    return _tc_tail_call(accp, denp, bias.reshape(1, D),
                         ln_g.reshape(1, D), ln_b.reshape(1, D))


# speedup vs baseline: 18.9421x; 18.9421x over previous
import jax
import jax.numpy as jnp
from jax import lax
from jax.experimental import pallas as pl
from jax.experimental.pallas import tpu as pltpu
from jax.experimental.pallas import tpu_sc as plsc

N = 10000
D = 128
DH = 64
E = 320000

NC = 2
NS = 16
L = 16
EPT = E // NS
C = 80
NH = 10
NCHUNK = EPT // (NH * C)
SEG = NS * NH
NW = NC * NS
SEGW = SEG // NW
RPT = 624
TB = NS * RPT
TR = N - TB
DW = 16

_BA = 1000
_BC = 1000


def _tc_head(x_ref, w_ref, al_ref, ar_ref, hlo_ref, hhi_ref, elr_ref):
    h = jnp.dot(x_ref[...], w_ref[...], preferred_element_type=jnp.float32)
    hlo_ref[...] = h[:, :DH]
    hhi_ref[...] = h[:, DH:]
    el = jnp.sum(h * al_ref[...], axis=1, keepdims=True)
    er = jnp.sum(h * ar_ref[...], axis=1, keepdims=True)
    elr_ref[...] = jnp.concatenate([el, er], axis=1)


def _tc_head_call(x, W, al, ar):
    return pl.pallas_call(
        _tc_head,
        grid=(N // _BA,),
        in_specs=[
            pl.BlockSpec((_BA, D), lambda i: (i, 0)),
            pl.BlockSpec((D, D), lambda i: (0, 0)),
            pl.BlockSpec((1, D), lambda i: (0, 0)),
            pl.BlockSpec((1, D), lambda i: (0, 0)),
        ],
        out_specs=[
            pl.BlockSpec((_BA, DH), lambda i: (i, 0)),
            pl.BlockSpec((_BA, DH), lambda i: (i, 0)),
            pl.BlockSpec((_BA, 2), lambda i: (i, 0)),
        ],
        out_shape=[
            jax.ShapeDtypeStruct((N, DH), jnp.float32),
            jax.ShapeDtypeStruct((N, DH), jnp.float32),
            jax.ShapeDtypeStruct((N, 2), jnp.float32),
        ],
    )(x, W, al, ar)


def _sc_w(elr_hbm, src_hbm, dst_hbm, w_hbm, elr_v, src_v, dst_v, wseg_v):
    c = lax.axis_index("c")
    s = lax.axis_index("s")
    wid = c * NS + s

    pltpu.sync_copy(elr_hbm, elr_v)

    col0 = jnp.zeros((L,), jnp.int32)
    col1 = jnp.full((L,), 1, dtype=jnp.int32)

    def seg_body(q, carry):
        seg = wid * SEGW + q
        pltpu.sync_copy(src_hbm.at[seg], src_v)
        pltpu.sync_copy(dst_hbm.at[seg], dst_v)

        def chunk_body(j, carry2):
            def w_body(k, carry3):
                srcv = src_v[j, pl.ds(k * L, L)]
                dstv = dst_v[j, pl.ds(k * L, L)]
                e = (plsc.load_gather(elr_v, [srcv, col0])
                     + plsc.load_gather(elr_v, [dstv, col1]))
                e = jnp.where(e >= 0.0, e, e * 0.2)
                wseg_v[j, pl.ds(k * L, L)] = jnp.exp(e)
                return carry3

            lax.fori_loop(0, C // L, w_body, 0)
            return carry2

        lax.fori_loop(0, NCHUNK, chunk_body, 0)
        pltpu.sync_copy(wseg_v, w_hbm.at[seg])
        return carry

    lax.fori_loop(0, SEGW, seg_body, 0)


def _sc_w_call(elr, src3, dst3):
    mesh = plsc.VectorSubcoreMesh(
        core_axis_name="c", subcore_axis_name="s", num_cores=NC,
        num_subcores=NS)
    return pl.kernel(
        _sc_w,
        compiler_params=pltpu.CompilerParams(
            needs_layout_passes=False, use_tc_tiling_on_sc=False),
        out_type=jax.ShapeDtypeStruct((SEG, NCHUNK, C), jnp.float32),
        mesh=mesh,
        scratch_types=[
            pltpu.VMEM((N, 2), jnp.float32),
            pltpu.VMEM((NCHUNK, C), jnp.int32),
            pltpu.VMEM((NCHUNK, C), jnp.int32),
            pltpu.VMEM((NCHUNK, C), jnp.float32),
        ],
    )(elr, src3, dst3)


def _sc_edges(hlo_hbm, hhi_hbm, w_hbm, src_hbm, dst_hbm, z64_hbm, z16_hbm,
              accp_hbm, denp_hbm,
              src_v, dst_v, wseg_v, w_v, rows_v, acc_sh, den_sh, sem):
    c = lax.axis_index("c")
    s = lax.axis_index("s")

    pltpu.sync_copy(z16_hbm.at[pl.ds(0, C)], w_v)

    rbase = s * RPT
    pltpu.sync_copy(z64_hbm, acc_sh.at[pl.ds(rbase, RPT)])
    pltpu.sync_copy(z16_hbm, den_sh.at[pl.ds(rbase, RPT)])

    @pl.when(s == NS - 1)
    def _zero_tail():
        pltpu.sync_copy(z64_hbm.at[pl.ds(0, TR)], acc_sh.at[pl.ds(TB, TR)])
        pltpu.sync_copy(z16_hbm.at[pl.ds(0, TR)], den_sh.at[pl.ds(TB, TR)])

    plsc.subcore_barrier()

    col0 = jnp.zeros((L,), jnp.int32)

    def make_chunk_body(h_ref):
        def chunk_body(j, carry):
            gather = pltpu.async_copy(h_ref.at[src_v.at[j]], rows_v, sem)

            def w_body(k, carry2):
                w = wseg_v[j, pl.ds(k * L, L)]
                plsc.store_scatter(
                    w_v, [k * L + lax.iota(jnp.int32, L), col0], w)
                return carry2

            lax.fori_loop(0, C // L, w_body, 0)
            gather.wait()

            def row_body(r, carry2):
                wr = plsc.load_gather(
                    wseg_v, [jnp.full((L,), j, dtype=jnp.int32),
                             jnp.full((L,), r, dtype=jnp.int32)])
                for q in range(DH // L):
                    rows_v[r, pl.ds(q * L, L)] = rows_v[r, pl.ds(q * L, L)] * wr
                return carry2

            lax.fori_loop(0, C, row_body, 0)

            pltpu.sync_copy(rows_v, acc_sh.at[dst_v.at[j]], add=True)
            pltpu.sync_copy(w_v, den_sh.at[dst_v.at[j]], add=True)
            return carry

        return chunk_body

    def make_run(h_ref):
        body = make_chunk_body(h_ref)

        def seg_body(hh, carry):
            seg = s * NH + hh
            pltpu.sync_copy(src_hbm.at[seg], src_v)
            pltpu.sync_copy(dst_hbm.at[seg], dst_v)
            pltpu.sync_copy(w_hbm.at[seg], wseg_v)
            lax.fori_loop(0, NCHUNK, body, 0)
            return carry

        return seg_body

    @pl.when(c == 0)
    def _run_lo():
        lax.fori_loop(0, NH, make_run(hlo_hbm), 0)

    @pl.when(c == 1)
    def _run_hi():
        lax.fori_loop(0, NH, make_run(hhi_hbm), 0)

    plsc.subcore_barrier()
    pltpu.sync_copy(acc_sh.at[pl.ds(rbase, RPT)],
                    accp_hbm.at[c, pl.ds(rbase, RPT)])
    pltpu.sync_copy(den_sh.at[pl.ds(rbase, RPT)],
                    denp_hbm.at[c, pl.ds(rbase, RPT)])

    @pl.when(s == NS - 1)
    def _out_tail():
        pltpu.sync_copy(acc_sh.at[pl.ds(TB, TR)], accp_hbm.at[c, pl.ds(TB, TR)])
        pltpu.sync_copy(den_sh.at[pl.ds(TB, TR)], denp_hbm.at[c, pl.ds(TB, TR)])


def _sc_edges_call(hlo, hhi, w3, src3, dst3, z64, z16):
    mesh = plsc.VectorSubcoreMesh(
        core_axis_name="c", subcore_axis_name="s", num_cores=NC,
        num_subcores=NS)
    return pl.kernel(
        _sc_edges,
        compiler_params=pltpu.CompilerParams(
            needs_layout_passes=False, use_tc_tiling_on_sc=False),
        out_type=[
            jax.ShapeDtypeStruct((NC, N, DH), jnp.float32),
            jax.ShapeDtypeStruct((NC, N, DW), jnp.float32),
        ],
        mesh=mesh,
        scratch_types=[
            pltpu.VMEM((NCHUNK, C), jnp.int32),
            pltpu.VMEM((NCHUNK, C), jnp.int32),
            pltpu.VMEM((NCHUNK, C), jnp.float32),
            pltpu.VMEM((C, DW), jnp.float32),
            pltpu.VMEM((C, DH), jnp.float32),
            pltpu.VMEM_SHARED((N, DH), jnp.float32),
            pltpu.VMEM_SHARED((N, DW), jnp.float32),
            pltpu.SemaphoreType.DMA,
        ],
    )(hlo, hhi, w3, src3, dst3, z64, z16)


def _tc_tail(accp_ref, denp_ref, bias_ref, g_ref, b_ref, out_ref):
    acc = jnp.concatenate([accp_ref[0], accp_ref[1]], axis=1)
    den = denp_ref[0, :, 0:1]
    den = jnp.where(den > 0.0, den, 1.0)
    rst = acc / den + bias_ref[...]
    mu = jnp.mean(rst, axis=1, keepdims=True)
    var = jnp.mean((rst - mu) ** 2, axis=1, keepdims=True)
    y = (rst - mu) * lax.rsqrt(var + 1e-5) * g_ref[...] + b_ref[...]
    out_ref[...] = jnp.where(y > 0.0, y, jnp.exp(y) - 1.0)


def _tc_tail_call(accp, denp, bias, ln_g, ln_b):
    return pl.pallas_call(
        _tc_tail,
        grid=(N // _BC,),
        in_specs=[
            pl.BlockSpec((NC, _BC, DH), lambda i: (0, i, 0)),
            pl.BlockSpec((NC, _BC, DW), lambda i: (0, i, 0)),
            pl.BlockSpec((1, D), lambda i: (0, 0)),
            pl.BlockSpec((1, D), lambda i: (0, 0)),
            pl.BlockSpec((1, D), lambda i: (0, 0)),
        ],
        out_specs=pl.BlockSpec((_BC, D), lambda i: (i, 0)),
        out_shape=jax.ShapeDtypeStruct((N, D), jnp.float32),
    )(accp, denp, bias, ln_g, ln_b)


@jax.jit
def kernel(features, edge_index, W, attn_l, attn_r, bias, ln_g, ln_b):
    src = edge_index[0].astype(jnp.int32).reshape(SEG, NCHUNK, C)
    dst = edge_index[1].astype(jnp.int32).reshape(SEG, NCHUNK, C)
    al = attn_l.reshape(1, D).astype(jnp.float32)
    ar = attn_r.reshape(1, D).astype(jnp.float32)
    hlo, hhi, elr = _tc_head_call(features, W, al, ar)
    w3 = _sc_w_call(elr, src, dst)
    z64 = jnp.zeros((RPT, DH), jnp.float32)
    z16 = jnp.zeros((RPT, DW), jnp.float32)
    accp, denp = _sc_edges_call(hlo, hhi, w3, src, dst, z64, z16)
    return _tc_tail_call(accp, denp, bias.reshape(1, D),
                         ln_g.reshape(1, D), ln_b.reshape(1, D))


# speedup vs baseline: 28.4454x; 1.5017x over previous
import jax
import jax.numpy as jnp
from jax import lax
from jax.experimental import pallas as pl
from jax.experimental.pallas import tpu as pltpu
from jax.experimental.pallas import tpu_sc as plsc

N = 10000
D = 128
DH = 64
E = 320000

NC = 2
NS = 16
L = 16
EPT = E // NS
C = 80
NH = 10
NCHUNK = EPT // (NH * C)
SEG = NS * NH
NW = NC * NS
SEGW = SEG // NW
RPT = 624
TB = NS * RPT
TR = N - TB
DW = 16

_BA = 1000
_BC = 1000


def _tc_head(x_ref, w_ref, al_ref, ar_ref, hlo_ref, hhi_ref, elr_ref):
    h = jnp.dot(x_ref[...], w_ref[...], preferred_element_type=jnp.float32)
    hlo_ref[...] = h[:, :DH]
    hhi_ref[...] = h[:, DH:]
    el = jnp.sum(h * al_ref[...], axis=1, keepdims=True)
    er = jnp.sum(h * ar_ref[...], axis=1, keepdims=True)
    elr_ref[...] = jnp.concatenate([el, er], axis=1)


def _tc_head_call(x, W, al, ar):
    return pl.pallas_call(
        _tc_head,
        grid=(N // _BA,),
        in_specs=[
            pl.BlockSpec((_BA, D), lambda i: (i, 0)),
            pl.BlockSpec((D, D), lambda i: (0, 0)),
            pl.BlockSpec((1, D), lambda i: (0, 0)),
            pl.BlockSpec((1, D), lambda i: (0, 0)),
        ],
        out_specs=[
            pl.BlockSpec((_BA, DH), lambda i: (i, 0)),
            pl.BlockSpec((_BA, DH), lambda i: (i, 0)),
            pl.BlockSpec((_BA, 2), lambda i: (i, 0)),
        ],
        out_shape=[
            jax.ShapeDtypeStruct((N, DH), jnp.float32),
            jax.ShapeDtypeStruct((N, DH), jnp.float32),
            jax.ShapeDtypeStruct((N, 2), jnp.float32),
        ],
    )(x, W, al, ar)


def _sc_w(elr_hbm, src_hbm, dst_hbm, w_hbm, elr_v, src_v, dst_v, wseg_v):
    c = lax.axis_index("c")
    s = lax.axis_index("s")
    wid = c * NS + s

    pltpu.sync_copy(elr_hbm, elr_v)

    col0 = jnp.zeros((L,), jnp.int32)
    col1 = jnp.full((L,), 1, dtype=jnp.int32)

    def seg_body(q, carry):
        seg = wid * SEGW + q
        pltpu.sync_copy(src_hbm.at[seg], src_v)
        pltpu.sync_copy(dst_hbm.at[seg], dst_v)

        def chunk_body(j, carry2):
            def w_body(k, carry3):
                srcv = src_v[j, pl.ds(k * L, L)]
                dstv = dst_v[j, pl.ds(k * L, L)]
                e = (plsc.load_gather(elr_v, [srcv, col0])
                     + plsc.load_gather(elr_v, [dstv, col1]))
                e = jnp.where(e >= 0.0, e, e * 0.2)
                wseg_v[j, pl.ds(k * L, L)] = jnp.exp(e)
                return carry3

            lax.fori_loop(0, C // L, w_body, 0)
            return carry2

        lax.fori_loop(0, NCHUNK, chunk_body, 0)
        pltpu.sync_copy(wseg_v, w_hbm.at[seg])
        return carry

    lax.fori_loop(0, SEGW, seg_body, 0)


def _sc_w_call(elr, src3, dst3):
    mesh = plsc.VectorSubcoreMesh(
        core_axis_name="c", subcore_axis_name="s", num_cores=NC,
        num_subcores=NS)
    return pl.kernel(
        _sc_w,
        compiler_params=pltpu.CompilerParams(
            needs_layout_passes=False, use_tc_tiling_on_sc=False),
        out_type=jax.ShapeDtypeStruct((SEG, NCHUNK, C), jnp.float32),
        mesh=mesh,
        scratch_types=[
            pltpu.VMEM((N, 2), jnp.float32),
            pltpu.VMEM((NCHUNK, C), jnp.int32),
            pltpu.VMEM((NCHUNK, C), jnp.int32),
            pltpu.VMEM((NCHUNK, C), jnp.float32),
        ],
    )(elr, src3, dst3)


def _sc_edges(hlo_hbm, hhi_hbm, w_hbm, src_hbm, dst_hbm, z64_hbm, z16_hbm,
              accp_hbm, denp_hbm,
              src_v, dst_v, wseg_v, w_v, rows_v, acc_sh, den_sh,
              sem_g, sem_s):
    c = lax.axis_index("c")
    s = lax.axis_index("s")

    pltpu.sync_copy(z16_hbm.at[pl.ds(0, C)], w_v.at[0])
    pltpu.sync_copy(z16_hbm.at[pl.ds(0, C)], w_v.at[1])

    rbase = s * RPT
    pltpu.sync_copy(z64_hbm, acc_sh.at[pl.ds(rbase, RPT)])
    pltpu.sync_copy(z16_hbm, den_sh.at[pl.ds(rbase, RPT)])

    @pl.when(s == NS - 1)
    def _zero_tail():
        pltpu.sync_copy(z64_hbm.at[pl.ds(0, TR)], acc_sh.at[pl.ds(TB, TR)])
        pltpu.sync_copy(z16_hbm.at[pl.ds(0, TR)], den_sh.at[pl.ds(TB, TR)])

    plsc.subcore_barrier()

    col0 = jnp.zeros((L,), jnp.int32)

    def spread_w(jc, bb):
        def w_body(k, carry):
            w = wseg_v[jc, pl.ds(k * L, L)]
            plsc.store_scatter(
                w_v, [jnp.full((L,), bb, dtype=jnp.int32),
                      k * L + lax.iota(jnp.int32, L), col0], w)
            return carry

        lax.fori_loop(0, C // L, w_body, 0)

    def make_chunk_body(h_ref):
        def chunk_body(j, carry):
            b = j % 2
            nb = 1 - b
            pltpu.make_async_copy(
                h_ref.at[src_v.at[j]], rows_v.at[b], sem_g).wait()

            @pl.when(j > 0)
            def _drain_prev():
                pltpu.make_async_copy(
                    rows_v.at[nb], acc_sh.at[dst_v.at[j - 1]], sem_s).wait()
                pltpu.make_async_copy(
                    w_v.at[nb], den_sh.at[dst_v.at[j - 1]], sem_s).wait()

            @pl.when(j + 1 < NCHUNK)
            def _prefetch():
                pltpu.async_copy(
                    h_ref.at[src_v.at[j + 1]], rows_v.at[nb], sem_g)

            def row_body(r, carry2):
                wr = plsc.load_gather(
                    wseg_v, [jnp.full((L,), j, dtype=jnp.int32),
                             jnp.full((L,), r, dtype=jnp.int32)])
                for q in range(DH // L):
                    rows_v[b, r, pl.ds(q * L, L)] = (
                        rows_v[b, r, pl.ds(q * L, L)] * wr)
                return carry2

            lax.fori_loop(0, C, row_body, 0)

            @pl.when(j + 1 < NCHUNK)
            def _spread_next():
                spread_w(j + 1, nb)

            pltpu.async_copy(rows_v.at[b], acc_sh.at[dst_v.at[j]], sem_s,
                             add=True)
            pltpu.async_copy(w_v.at[b], den_sh.at[dst_v.at[j]], sem_s,
                             add=True)
            return carry

        return chunk_body

    def make_run(h_ref):
        body = make_chunk_body(h_ref)
        lb = (NCHUNK - 1) % 2

        def seg_body(hh, carry):
            seg = s * NH + hh
            pltpu.sync_copy(src_hbm.at[seg], src_v)
            pltpu.sync_copy(dst_hbm.at[seg], dst_v)
            pltpu.sync_copy(w_hbm.at[seg], wseg_v)
            spread_w(0, 0)
            pltpu.async_copy(h_ref.at[src_v.at[0]], rows_v.at[0], sem_g)
            lax.fori_loop(0, NCHUNK, body, 0)
            pltpu.make_async_copy(
                rows_v.at[lb], acc_sh.at[dst_v.at[NCHUNK - 1]], sem_s).wait()
            pltpu.make_async_copy(
                w_v.at[lb], den_sh.at[dst_v.at[NCHUNK - 1]], sem_s).wait()
            return carry

        return seg_body

    @pl.when(c == 0)
    def _run_lo():
        lax.fori_loop(0, NH, make_run(hlo_hbm), 0)

    @pl.when(c == 1)
    def _run_hi():
        lax.fori_loop(0, NH, make_run(hhi_hbm), 0)

    plsc.subcore_barrier()
    pltpu.sync_copy(acc_sh.at[pl.ds(rbase, RPT)],
                    accp_hbm.at[c, pl.ds(rbase, RPT)])
    pltpu.sync_copy(den_sh.at[pl.ds(rbase, RPT)],
                    denp_hbm.at[c, pl.ds(rbase, RPT)])

    @pl.when(s == NS - 1)
    def _out_tail():
        pltpu.sync_copy(acc_sh.at[pl.ds(TB, TR)], accp_hbm.at[c, pl.ds(TB, TR)])
        pltpu.sync_copy(den_sh.at[pl.ds(TB, TR)], denp_hbm.at[c, pl.ds(TB, TR)])


def _sc_edges_call(hlo, hhi, w3, src3, dst3, z64, z16):
    mesh = plsc.VectorSubcoreMesh(
        core_axis_name="c", subcore_axis_name="s", num_cores=NC,
        num_subcores=NS)
    return pl.kernel(
        _sc_edges,
        compiler_params=pltpu.CompilerParams(
            needs_layout_passes=False, use_tc_tiling_on_sc=False),
        out_type=[
            jax.ShapeDtypeStruct((NC, N, DH), jnp.float32),
            jax.ShapeDtypeStruct((NC, N, DW), jnp.float32),
        ],
        mesh=mesh,
        scratch_types=[
            pltpu.VMEM((NCHUNK, C), jnp.int32),
            pltpu.VMEM((NCHUNK, C), jnp.int32),
            pltpu.VMEM((NCHUNK, C), jnp.float32),
            pltpu.VMEM((2, C, DW), jnp.float32),
            pltpu.VMEM((2, C, DH), jnp.float32),
            pltpu.VMEM_SHARED((N, DH), jnp.float32),
            pltpu.VMEM_SHARED((N, DW), jnp.float32),
            pltpu.SemaphoreType.DMA,
            pltpu.SemaphoreType.DMA,
        ],
    )(hlo, hhi, w3, src3, dst3, z64, z16)


def _tc_tail(accp_ref, denp_ref, bias_ref, g_ref, b_ref, out_ref):
    acc = jnp.concatenate([accp_ref[0], accp_ref[1]], axis=1)
    den = denp_ref[0, :, 0:1]
    den = jnp.where(den > 0.0, den, 1.0)
    rst = acc / den + bias_ref[...]
    mu = jnp.mean(rst, axis=1, keepdims=True)
    var = jnp.mean((rst - mu) ** 2, axis=1, keepdims=True)
    y = (rst - mu) * lax.rsqrt(var + 1e-5) * g_ref[...] + b_ref[...]
    out_ref[...] = jnp.where(y > 0.0, y, jnp.exp(y) - 1.0)


def _tc_tail_call(accp, denp, bias, ln_g, ln_b):
    return pl.pallas_call(
        _tc_tail,
        grid=(N // _BC,),
        in_specs=[
            pl.BlockSpec((NC, _BC, DH), lambda i: (0, i, 0)),
            pl.BlockSpec((NC, _BC, DW), lambda i: (0, i, 0)),
            pl.BlockSpec((1, D), lambda i: (0, 0)),
            pl.BlockSpec((1, D), lambda i: (0, 0)),
            pl.BlockSpec((1, D), lambda i: (0, 0)),
        ],
        out_specs=pl.BlockSpec((_BC, D), lambda i: (i, 0)),
        out_shape=jax.ShapeDtypeStruct((N, D), jnp.float32),
    )(accp, denp, bias, ln_g, ln_b)


@jax.jit
def kernel(features, edge_index, W, attn_l, attn_r, bias, ln_g, ln_b):
    src = edge_index[0].astype(jnp.int32).reshape(SEG, NCHUNK, C)
    dst = edge_index[1].astype(jnp.int32).reshape(SEG, NCHUNK, C)
    al = attn_l.reshape(1, D).astype(jnp.float32)
    ar = attn_r.reshape(1, D).astype(jnp.float32)
    hlo, hhi, elr = _tc_head_call(features, W, al, ar)
    w3 = _sc_w_call(elr, src, dst)
    z64 = jnp.zeros((RPT, DH), jnp.float32)
    z16 = jnp.zeros((RPT, DW), jnp.float32)
    accp, denp = _sc_edges_call(hlo, hhi, w3, src, dst, z64, z16)
    return _tc_tail_call(accp, denp, bias.reshape(1, D),
                         ln_g.reshape(1, D), ln_b.reshape(1, D))


# speedup vs baseline: 31.0470x; 1.0915x over previous
import jax
import jax.numpy as jnp
from jax import lax
from jax.experimental import pallas as pl
from jax.experimental.pallas import tpu as pltpu
from jax.experimental.pallas import tpu_sc as plsc

N = 10000
D = 128
DH = 64
E = 320000

NC = 2
NS = 16
L = 16
EPT = E // NS
C = 80
NH = 10
NCHUNK = EPT // (NH * C)
SEG = NS * NH
NW = NC * NS
SEGW = SEG // NW
RPT = 624
TB = NS * RPT
TR = N - TB
DW = 16

_BA = 1000
_BC = 1000


def _tc_head(x_ref, w_ref, al_ref, ar_ref, hlo_ref, hhi_ref, elr_ref):
    h = jnp.dot(x_ref[...], w_ref[...], preferred_element_type=jnp.float32)
    hlo_ref[...] = h[:, :DH]
    hhi_ref[...] = h[:, DH:]
    el = jnp.sum(h * al_ref[...], axis=1, keepdims=True)
    er = jnp.sum(h * ar_ref[...], axis=1, keepdims=True)
    elr_ref[...] = jnp.concatenate([el, er], axis=1)


def _tc_head_call(x, W, al, ar):
    return pl.pallas_call(
        _tc_head,
        grid=(N // _BA,),
        in_specs=[
            pl.BlockSpec((_BA, D), lambda i: (i, 0)),
            pl.BlockSpec((D, D), lambda i: (0, 0)),
            pl.BlockSpec((1, D), lambda i: (0, 0)),
            pl.BlockSpec((1, D), lambda i: (0, 0)),
        ],
        out_specs=[
            pl.BlockSpec((_BA, DH), lambda i: (i, 0)),
            pl.BlockSpec((_BA, DH), lambda i: (i, 0)),
            pl.BlockSpec((_BA, 2), lambda i: (i, 0)),
        ],
        out_shape=[
            jax.ShapeDtypeStruct((N, DH), jnp.float32),
            jax.ShapeDtypeStruct((N, DH), jnp.float32),
            jax.ShapeDtypeStruct((N, 2), jnp.float32),
        ],
    )(x, W, al, ar)


def _sc_w(elr_hbm, src_hbm, dst_hbm, w_hbm, elr_v, src_v, dst_v, wseg_v):
    c = lax.axis_index("c")
    s = lax.axis_index("s")
    wid = c * NS + s

    pltpu.sync_copy(elr_hbm, elr_v)

    col0 = jnp.zeros((L,), jnp.int32)
    col1 = jnp.full((L,), 1, dtype=jnp.int32)

    def seg_body(q, carry):
        seg = wid * SEGW + q
        pltpu.sync_copy(src_hbm.at[seg], src_v)
        pltpu.sync_copy(dst_hbm.at[seg], dst_v)

        @plsc.parallel_loop(0, NCHUNK * (C // L), unroll=4)
        def w_body(m):
            j = m // (C // L)
            k = m % (C // L)
            srcv = src_v[j, pl.ds(k * L, L)]
            dstv = dst_v[j, pl.ds(k * L, L)]
            e = (plsc.load_gather(elr_v, [srcv, col0])
                 + plsc.load_gather(elr_v, [dstv, col1]))
            e = jnp.where(e >= 0.0, e, e * 0.2)
            wseg_v[j, pl.ds(k * L, L)] = jnp.exp(e)
        pltpu.sync_copy(wseg_v, w_hbm.at[seg])
        return carry

    lax.fori_loop(0, SEGW, seg_body, 0)


def _sc_w_call(elr, src3, dst3):
    mesh = plsc.VectorSubcoreMesh(
        core_axis_name="c", subcore_axis_name="s", num_cores=NC,
        num_subcores=NS)
    return pl.kernel(
        _sc_w,
        compiler_params=pltpu.CompilerParams(
            needs_layout_passes=False, use_tc_tiling_on_sc=False),
        out_type=jax.ShapeDtypeStruct((SEG, NCHUNK, C), jnp.float32),
        mesh=mesh,
        scratch_types=[
            pltpu.VMEM((N, 2), jnp.float32),
            pltpu.VMEM((NCHUNK, C), jnp.int32),
            pltpu.VMEM((NCHUNK, C), jnp.int32),
            pltpu.VMEM((NCHUNK, C), jnp.float32),
        ],
    )(elr, src3, dst3)


def _sc_edges(hlo_hbm, hhi_hbm, w_hbm, src_hbm, dst_hbm, z64_hbm, z16_hbm,
              accp_hbm, denp_hbm,
              src_v, dst_v, wseg_v, w_v, rows_v, acc_sh, den_sh,
              sem_g, sem_s):
    c = lax.axis_index("c")
    s = lax.axis_index("s")

    pltpu.sync_copy(z16_hbm.at[pl.ds(0, C)], w_v.at[0])
    pltpu.sync_copy(z16_hbm.at[pl.ds(0, C)], w_v.at[1])

    rbase = s * RPT
    pltpu.sync_copy(z64_hbm, acc_sh.at[pl.ds(rbase, RPT)])
    pltpu.sync_copy(z16_hbm, den_sh.at[pl.ds(rbase, RPT)])

    @pl.when(s == NS - 1)
    def _zero_tail():
        pltpu.sync_copy(z64_hbm.at[pl.ds(0, TR)], acc_sh.at[pl.ds(TB, TR)])
        pltpu.sync_copy(z16_hbm.at[pl.ds(0, TR)], den_sh.at[pl.ds(TB, TR)])

    plsc.subcore_barrier()

    col0 = jnp.zeros((L,), jnp.int32)

    def spread_w(jc, bb):
        @plsc.parallel_loop(0, C // L, unroll=C // L)
        def w_body(k):
            w = wseg_v[jc, pl.ds(k * L, L)]
            plsc.store_scatter(
                w_v, [jnp.full((L,), bb, dtype=jnp.int32),
                      k * L + lax.iota(jnp.int32, L), col0], w)

    def make_chunk_body(h_ref):
        def chunk_body(j, carry):
            b = j % 2
            nb = 1 - b
            pltpu.make_async_copy(
                h_ref.at[src_v.at[j]], rows_v.at[b], sem_g).wait()

            @pl.when(j > 0)
            def _drain_prev():
                pltpu.make_async_copy(
                    rows_v.at[nb], acc_sh.at[dst_v.at[j - 1]], sem_s).wait()
                pltpu.make_async_copy(
                    w_v.at[nb], den_sh.at[dst_v.at[j - 1]], sem_s).wait()

            @pl.when(j + 1 < NCHUNK)
            def _prefetch():
                pltpu.async_copy(
                    h_ref.at[src_v.at[j + 1]], rows_v.at[nb], sem_g)

            @plsc.parallel_loop(0, C, unroll=8)
            def row_body(r):
                wr = plsc.load_gather(
                    wseg_v, [jnp.full((L,), j, dtype=jnp.int32),
                             jnp.full((L,), r, dtype=jnp.int32)])
                for q in range(DH // L):
                    rows_v[b, r, pl.ds(q * L, L)] = (
                        rows_v[b, r, pl.ds(q * L, L)] * wr)

            @pl.when(j + 1 < NCHUNK)
            def _spread_next():
                spread_w(j + 1, nb)

            pltpu.async_copy(rows_v.at[b], acc_sh.at[dst_v.at[j]], sem_s,
                             add=True)
            pltpu.async_copy(w_v.at[b], den_sh.at[dst_v.at[j]], sem_s,
                             add=True)
            return carry

        return chunk_body

    def make_run(h_ref):
        body = make_chunk_body(h_ref)
        lb = (NCHUNK - 1) % 2

        def seg_body(hh, carry):
            seg = s * NH + hh
            pltpu.sync_copy(src_hbm.at[seg], src_v)
            pltpu.sync_copy(dst_hbm.at[seg], dst_v)
            pltpu.sync_copy(w_hbm.at[seg], wseg_v)
            spread_w(0, 0)
            pltpu.async_copy(h_ref.at[src_v.at[0]], rows_v.at[0], sem_g)
            lax.fori_loop(0, NCHUNK, body, 0)
            pltpu.make_async_copy(
                rows_v.at[lb], acc_sh.at[dst_v.at[NCHUNK - 1]], sem_s).wait()
            pltpu.make_async_copy(
                w_v.at[lb], den_sh.at[dst_v.at[NCHUNK - 1]], sem_s).wait()
            return carry

        return seg_body

    @pl.when(c == 0)
    def _run_lo():
        lax.fori_loop(0, NH, make_run(hlo_hbm), 0)

    @pl.when(c == 1)
    def _run_hi():
        lax.fori_loop(0, NH, make_run(hhi_hbm), 0)

    plsc.subcore_barrier()
    pltpu.sync_copy(acc_sh.at[pl.ds(rbase, RPT)],
                    accp_hbm.at[c, pl.ds(rbase, RPT)])
    pltpu.sync_copy(den_sh.at[pl.ds(rbase, RPT)],
                    denp_hbm.at[c, pl.ds(rbase, RPT)])

    @pl.when(s == NS - 1)
    def _out_tail():
        pltpu.sync_copy(acc_sh.at[pl.ds(TB, TR)], accp_hbm.at[c, pl.ds(TB, TR)])
        pltpu.sync_copy(den_sh.at[pl.ds(TB, TR)], denp_hbm.at[c, pl.ds(TB, TR)])


def _sc_edges_call(hlo, hhi, w3, src3, dst3, z64, z16):
    mesh = plsc.VectorSubcoreMesh(
        core_axis_name="c", subcore_axis_name="s", num_cores=NC,
        num_subcores=NS)
    return pl.kernel(
        _sc_edges,
        compiler_params=pltpu.CompilerParams(
            needs_layout_passes=False, use_tc_tiling_on_sc=False),
        out_type=[
            jax.ShapeDtypeStruct((NC, N, DH), jnp.float32),
            jax.ShapeDtypeStruct((NC, N, DW), jnp.float32),
        ],
        mesh=mesh,
        scratch_types=[
            pltpu.VMEM((NCHUNK, C), jnp.int32),
            pltpu.VMEM((NCHUNK, C), jnp.int32),
            pltpu.VMEM((NCHUNK, C), jnp.float32),
            pltpu.VMEM((2, C, DW), jnp.float32),
            pltpu.VMEM((2, C, DH), jnp.float32),
            pltpu.VMEM_SHARED((N, DH), jnp.float32),
            pltpu.VMEM_SHARED((N, DW), jnp.float32),
            pltpu.SemaphoreType.DMA,
            pltpu.SemaphoreType.DMA,
        ],
    )(hlo, hhi, w3, src3, dst3, z64, z16)


def _tc_tail(accp_ref, denp_ref, bias_ref, g_ref, b_ref, out_ref):
    acc = jnp.concatenate([accp_ref[0], accp_ref[1]], axis=1)
    den = denp_ref[0, :, 0:1]
    den = jnp.where(den > 0.0, den, 1.0)
    rst = acc / den + bias_ref[...]
    mu = jnp.mean(rst, axis=1, keepdims=True)
    var = jnp.mean((rst - mu) ** 2, axis=1, keepdims=True)
    y = (rst - mu) * lax.rsqrt(var + 1e-5) * g_ref[...] + b_ref[...]
    out_ref[...] = jnp.where(y > 0.0, y, jnp.exp(y) - 1.0)


def _tc_tail_call(accp, denp, bias, ln_g, ln_b):
    return pl.pallas_call(
        _tc_tail,
        grid=(N // _BC,),
        in_specs=[
            pl.BlockSpec((NC, _BC, DH), lambda i: (0, i, 0)),
            pl.BlockSpec((NC, _BC, DW), lambda i: (0, i, 0)),
            pl.BlockSpec((1, D), lambda i: (0, 0)),
            pl.BlockSpec((1, D), lambda i: (0, 0)),
            pl.BlockSpec((1, D), lambda i: (0, 0)),
        ],
        out_specs=pl.BlockSpec((_BC, D), lambda i: (i, 0)),
        out_shape=jax.ShapeDtypeStruct((N, D), jnp.float32),
    )(accp, denp, bias, ln_g, ln_b)


@jax.jit
def kernel(features, edge_index, W, attn_l, attn_r, bias, ln_g, ln_b):
    src = edge_index[0].astype(jnp.int32).reshape(SEG, NCHUNK, C)
    dst = edge_index[1].astype(jnp.int32).reshape(SEG, NCHUNK, C)
    al = attn_l.reshape(1, D).astype(jnp.float32)
    ar = attn_r.reshape(1, D).astype(jnp.float32)
    hlo, hhi, elr = _tc_head_call(features, W, al, ar)
    w3 = _sc_w_call(elr, src, dst)
    z64 = jnp.zeros((RPT, DH), jnp.float32)
    z16 = jnp.zeros((RPT, DW), jnp.float32)
    accp, denp = _sc_edges_call(hlo, hhi, w3, src, dst, z64, z16)
    return _tc_tail_call(accp, denp, bias.reshape(1, D),
                         ln_g.reshape(1, D), ln_b.reshape(1, D))
